# Initial kernel scaffold; baseline (speedup 1.0000x reference)
import jax
import jax.numpy as jnp
from jax.experimental import pallas as pl


def kernel(adjacency_matrix, x, hidden, h_h, h_c, Wf1, bf1, Wf2, bf2, W_ih, W_hh, b_ih, b_hh, Wo1, bo1, Wo2, bo2):
    raise NotImplementedError("write your pallas kernel here")


# speedup vs baseline: 1.3097x; 1.3097x over previous
import functools

import jax
import jax.numpy as jnp
from jax.experimental import pallas as pl

N = 512
D = 64
MSG = 64
F_HID = 128
CHUNK = 8
_HI = jax.lax.Precision.HIGHEST


def _bf(v):
    return v.astype(jnp.bfloat16)


def _ab_body(hid_ref, wf1a_ref, wf1b_ref, bf1_ref, a_ref, b_ref):
    hid = hid_ref[:]
    a_ref[:] = jnp.dot(hid, wf1a_ref[:], preferred_element_type=jnp.float32) + bf1_ref[:]
    b_ref[:] = jnp.dot(hid, wf1b_ref[:], preferred_element_type=jnp.float32)


def _sum_body(a3_ref, b_ref, adjt3_ref, s_ref):
    c = pl.program_id(0)

    @pl.when(c == 0)
    def _():
        s_ref[:] = jnp.zeros_like(s_ref)

    B = b_ref[:]
    a_chunk = a3_ref[0]
    m_chunk = adjt3_ref[0]
    acc = s_ref[:]
    for k in range(CHUNK):
        ak = a_chunk[k:k + 1, :]
        mk = m_chunk[:, k:k + 1]
        r = jnp.maximum(ak + B, 0.0).astype(jnp.bfloat16).astype(jnp.float32)
        acc = acc + r * mk
    s_ref[:] = acc


def _tail_body(s_ref, adjt_ref, x_ref, hprev_ref, cprev_ref,
               wf2t_ref, bf2_ref, wiht_ref, whht_ref, bsum_ref,
               wo1t_ref, bo1_ref, wo2t_ref, bo2_ref,
               out_ref, h_ref, c_ref):
    deg = jnp.sum(adjt_ref[:], axis=1, keepdims=True)
    msg = (jnp.dot(s_ref[:], wf2t_ref[:], preferred_element_type=jnp.float32,
                   precision=_HI)
           + deg * bf2_ref[:])

    inp = jnp.concatenate([x_ref[:], msg], axis=1).astype(jnp.bfloat16)
    gates = (jnp.dot(inp, wiht_ref[:], preferred_element_type=jnp.float32)
             + jnp.dot(hprev_ref[:], whht_ref[:], preferred_element_type=jnp.float32)
             + bsum_ref[:])
    i_g = jax.nn.sigmoid(gates[:, 0 * D:1 * D])
    f_g = jax.nn.sigmoid(gates[:, 1 * D:2 * D])
    g_g = jnp.tanh(gates[:, 2 * D:3 * D])
    o_g = jax.nn.sigmoid(gates[:, 3 * D:4 * D])
    c_new = f_g * cprev_ref[:] + i_g * g_g
    h_new = o_g * jnp.tanh(c_new)

    hid1 = jnp.maximum(
        jnp.dot(h_new.astype(jnp.bfloat16), wo1t_ref[:],
                preferred_element_type=jnp.float32) + bo1_ref[:], 0.0)
    out_ref[:] = (jnp.dot(hid1.astype(jnp.bfloat16), wo2t_ref[:],
                          preferred_element_type=jnp.float32) + bo2_ref[:])
    h_ref[:] = h_new
    c_ref[:] = c_new


@functools.partial(jax.jit, static_argnames=("interpret",))
def _run(adjacency_matrix, x, hidden, h_h, h_c, Wf1, bf1, Wf2, bf2,
         W_ih, W_hh, b_ih, b_hh, Wo1, bo1, Wo2, bo2, interpret=False):
    n, d = hidden.shape
    adjt = adjacency_matrix.T.astype(jnp.float32)
    adjt3 = adjt.reshape(n, n // CHUNK, CHUNK).transpose(1, 0, 2)

    A, B = pl.pallas_call(
        _ab_body,
        out_shape=[jax.ShapeDtypeStruct((n, F_HID), jnp.float32)] * 2,
        interpret=interpret,
    )(_bf(hidden), _bf(Wf1[:, :d].T), _bf(Wf1[:, d:].T), bf1[None, :])
    A3 = A.reshape(n // CHUNK, CHUNK, F_HID)

    S = pl.pallas_call(
        _sum_body,
        grid=(n // CHUNK,),
        in_specs=[
            pl.BlockSpec((1, CHUNK, F_HID), lambda c: (c, 0, 0)),
            pl.BlockSpec((n, F_HID), lambda c: (0, 0)),
            pl.BlockSpec((1, n, CHUNK), lambda c: (c, 0, 0)),
        ],
        out_specs=pl.BlockSpec((n, F_HID), lambda c: (0, 0)),
        out_shape=jax.ShapeDtypeStruct((n, F_HID), jnp.float32),
        interpret=interpret,
    )(A3, B, adjt3)

    out, h_new, c_new = pl.pallas_call(
        _tail_body,
        out_shape=[
            jax.ShapeDtypeStruct((n, Wo2.shape[0]), jnp.float32),
            jax.ShapeDtypeStruct((n, d), jnp.float32),
            jax.ShapeDtypeStruct((n, d), jnp.float32),
        ],
        interpret=interpret,
    )(S, adjt, x, _bf(h_h[0]), h_c[0],
      _bf(Wf2.T).astype(jnp.float32), bf2[None, :],
      _bf(W_ih.T), _bf(W_hh.T), (b_ih + b_hh)[None, :],
      _bf(Wo1.T), bo1[None, :], _bf(Wo2.T), bo2[None, :])
    return out, h_new, h_new[None, :, :], c_new[None, :, :]


def kernel(adjacency_matrix, x, hidden, h_h, h_c, Wf1, bf1, Wf2, bf2,
           W_ih, W_hh, b_ih, b_hh, Wo1, bo1, Wo2, bo2):
    return _run(adjacency_matrix, x, hidden, h_h, h_c, Wf1, bf1, Wf2, bf2,
                W_ih, W_hh, b_ih, b_hh, Wo1, bo1, Wo2, bo2)
